# baseline (speedup 1.0000x reference)
import functools
import math

import jax
import jax.numpy as jnp
from jax import lax
from jax.experimental import pallas as pl
from jax.experimental.pallas import tpu as pltpu
from jax.experimental.pallas import tpu_sc as plsc

D_MODEL = 512
LANES = 16
NUM_CORES = 2
NUM_SUBCORES = 16
NUM_WORKERS = NUM_CORES * NUM_SUBCORES
CHUNK = 16
NBUF = 14
AHEAD = 12


@functools.lru_cache(maxsize=None)
def _make_embedding_kernel(nrows: int, ncols: int):
    batch = nrows * ncols
    rows_per_worker = batch // NUM_WORKERS
    num_chunks = rows_per_worker // CHUNK
    assert num_chunks >= NBUF > AHEAD >= 1 and NBUF - AHEAD >= 2
    assert ncols % rows_per_worker == 0
    workers_per_row = ncols // rows_per_worker
    scale = math.sqrt(D_MODEL)
    mesh = plsc.VectorSubcoreMesh(core_axis_name="c", subcore_axis_name="s")

    @functools.partial(
        pl.kernel,
        out_type=jax.ShapeDtypeStruct((batch, D_MODEL), jnp.float32),
        mesh=mesh,
        scratch_types=[
            pltpu.VMEM((rows_per_worker,), jnp.int32),
            pltpu.VMEM((NBUF, CHUNK, D_MODEL), jnp.float32),
            pltpu.SemaphoreType.DMA((NBUF,)),
            pltpu.SemaphoreType.DMA((NBUF,)),
        ],
    )
    def emb_kernel(table_hbm, idx_hbm, out_hbm, idx_v, bufs, gsems, wsems):
        wid = lax.axis_index("s") * NUM_CORES + lax.axis_index("c")
        base = wid * rows_per_worker
        pltpu.sync_copy(
            idx_hbm.at[
                wid // workers_per_row,
                pl.ds((wid % workers_per_row) * rows_per_worker, rows_per_worker),
            ],
            idx_v,
        )

        def start_gather(c, b):
            pltpu.async_copy(
                table_hbm.at[idx_v.at[pl.ds(c * CHUNK, CHUNK)]],
                bufs.at[b],
                gsems.at[b],
            )

        def wait_gather(c, b):
            pltpu.make_async_copy(
                table_hbm.at[idx_v.at[pl.ds(c * CHUNK, CHUNK)]],
                bufs.at[b],
                gsems.at[b],
            ).wait()

        def start_wb(c, b):
            pltpu.async_copy(
                bufs.at[b],
                out_hbm.at[pl.ds(base + c * CHUNK, CHUNK)],
                wsems.at[b],
            )

        def wait_wb(c, b):
            pltpu.make_async_copy(
                bufs.at[b],
                out_hbm.at[pl.ds(base + c * CHUNK, CHUNK)],
                wsems.at[b],
            ).wait()

        @pl.loop(0, AHEAD)
        def _(c):
            start_gather(c, c)

        @pl.loop(0, num_chunks)
        def _(c):
            b = lax.rem(c, NBUF)
            wait_gather(c, b)
            nc = c + AHEAD

            @pl.when(nc < num_chunks)
            def _():
                b2 = lax.rem(nc, NBUF)

                @pl.when(nc >= NBUF)
                def _():
                    wait_wb(nc - NBUF, b2)

                start_gather(nc, b2)

            half = CHUNK // 2
            for h in range(2):

                @pl.loop(h * half, (h + 1) * half)
                def _(r):
                    for c0 in range(0, D_MODEL, LANES):
                        bufs[b, r, pl.ds(c0, LANES)] = (
                            bufs[b, r, pl.ds(c0, LANES)] * scale
                        )

                pltpu.async_copy(
                    bufs.at[b, pl.ds(h * half, half)],
                    out_hbm.at[pl.ds(base + c * CHUNK + h * half, half)],
                    wsems.at[b],
                )

        @pl.loop(num_chunks - NBUF, num_chunks)
        def _(c):
            wait_wb(c, lax.rem(c, NBUF))

    return emb_kernel


@jax.jit
def kernel(x, table):
    b, s = x.shape
    out = _make_embedding_kernel(b, s)(table, x.astype(jnp.int32))
    return out.reshape(b, s, D_MODEL)


# speedup vs baseline: 2.7729x; 2.7729x over previous
import functools
import math

import jax
import jax.numpy as jnp
from jax import lax
from jax.experimental import pallas as pl
from jax.experimental.pallas import tpu as pltpu
from jax.experimental.pallas import tpu_sc as plsc

D_MODEL = 512
LANES = 16
NUM_CORES = 2
NUM_SUBCORES = 16
NUM_WORKERS = NUM_CORES * NUM_SUBCORES
CHUNK = 32
NBUF = 7
AHEAD = 5


@functools.lru_cache(maxsize=None)
def _make_embedding_kernel(nrows: int, ncols: int):
    batch = nrows * ncols
    rows_per_worker = batch // NUM_WORKERS
    num_chunks = rows_per_worker // CHUNK
    assert num_chunks >= NBUF > AHEAD >= 1 and NBUF - AHEAD >= 2
    assert ncols % rows_per_worker == 0
    workers_per_row = ncols // rows_per_worker
    scale = math.sqrt(D_MODEL)
    mesh = plsc.VectorSubcoreMesh(core_axis_name="c", subcore_axis_name="s")

    @functools.partial(
        pl.kernel,
        out_type=jax.ShapeDtypeStruct((batch, D_MODEL), jnp.float32),
        mesh=mesh,
        scratch_types=[
            pltpu.VMEM((rows_per_worker,), jnp.int32),
            pltpu.VMEM((NBUF, CHUNK, D_MODEL), jnp.float32),
            pltpu.SemaphoreType.DMA((NBUF,)),
            pltpu.SemaphoreType.DMA((NBUF,)),
        ],
    )
    def emb_kernel(table_hbm, idx_hbm, out_hbm, idx_v, bufs, gsems, wsems):
        wid = lax.axis_index("s") * NUM_CORES + lax.axis_index("c")
        base = wid * rows_per_worker
        pltpu.sync_copy(
            idx_hbm.at[
                wid // workers_per_row,
                pl.ds((wid % workers_per_row) * rows_per_worker, rows_per_worker),
            ],
            idx_v,
        )

        def start_gather(c, b):
            pltpu.async_copy(
                table_hbm.at[idx_v.at[pl.ds(c * CHUNK, CHUNK)]],
                bufs.at[b],
                gsems.at[b],
            )

        def wait_gather(c, b):
            pltpu.make_async_copy(
                table_hbm.at[idx_v.at[pl.ds(c * CHUNK, CHUNK)]],
                bufs.at[b],
                gsems.at[b],
            ).wait()

        def start_wb(c, b):
            pltpu.async_copy(
                bufs.at[b],
                out_hbm.at[pl.ds(base + c * CHUNK, CHUNK)],
                wsems.at[b],
            )

        def wait_wb(c, b):
            pltpu.make_async_copy(
                bufs.at[b],
                out_hbm.at[pl.ds(base + c * CHUNK, CHUNK)],
                wsems.at[b],
            ).wait()

        @pl.loop(0, AHEAD)
        def _(c):
            start_gather(c, c)

        @pl.loop(0, num_chunks)
        def _(c):
            b = lax.rem(c, NBUF)
            wait_gather(c, b)
            nc = c + AHEAD

            @pl.when(nc < num_chunks)
            def _():
                b2 = lax.rem(nc, NBUF)

                @pl.when(nc >= NBUF)
                def _():
                    wait_wb(nc - NBUF, b2)

                start_gather(nc, b2)

            half = CHUNK // 2
            for h in range(2):

                @pl.loop(h * half, (h + 1) * half)
                def _(r):
                    for c0 in range(0, D_MODEL, LANES):
                        bufs[b, r, pl.ds(c0, LANES)] = (
                            bufs[b, r, pl.ds(c0, LANES)] * scale
                        )

                pltpu.async_copy(
                    bufs.at[b, pl.ds(h * half, half)],
                    out_hbm.at[pl.ds(base + c * CHUNK + h * half, half)],
                    wsems.at[b],
                )

        @pl.loop(num_chunks - NBUF, num_chunks)
        def _(c):
            wait_wb(c, lax.rem(c, NBUF))

    return emb_kernel


@jax.jit
def kernel(x, table):
    b, s = x.shape
    out = _make_embedding_kernel(b, s)(table, x.astype(jnp.int32))
    return out.reshape(b, s, D_MODEL)


# speedup vs baseline: 2.7913x; 1.0067x over previous
import functools
import math

import jax
import jax.numpy as jnp
from jax import lax
from jax.experimental import pallas as pl
from jax.experimental.pallas import tpu as pltpu
from jax.experimental.pallas import tpu_sc as plsc

D_MODEL = 512
LANES = 16
NUM_CORES = 2
NUM_SUBCORES = 16
NUM_WORKERS = NUM_CORES * NUM_SUBCORES
CHUNK = 32
NBUF = 7
AHEAD = 5


@functools.lru_cache(maxsize=None)
def _make_embedding_kernel(nrows: int, ncols: int):
    batch = nrows * ncols
    rows_per_worker = batch // NUM_WORKERS
    num_chunks = rows_per_worker // CHUNK
    assert num_chunks >= NBUF > AHEAD >= 1 and NBUF - AHEAD >= 2
    assert ncols % rows_per_worker == 0
    workers_per_row = ncols // rows_per_worker
    scale = math.sqrt(D_MODEL)
    mesh = plsc.VectorSubcoreMesh(core_axis_name="c", subcore_axis_name="s")

    @functools.partial(
        pl.kernel,
        out_type=jax.ShapeDtypeStruct((batch, D_MODEL), jnp.float32),
        mesh=mesh,
        scratch_types=[
            pltpu.VMEM((rows_per_worker,), jnp.int32),
            pltpu.VMEM((NBUF, CHUNK, D_MODEL), jnp.float32),
            pltpu.SemaphoreType.DMA((NBUF,)),
            pltpu.SemaphoreType.DMA((NBUF,)),
        ],
    )
    def emb_kernel(table_hbm, idx_hbm, out_hbm, idx_v, bufs, gsems, wsems):
        wid = lax.axis_index("s") * NUM_CORES + lax.axis_index("c")
        base = wid * rows_per_worker
        pltpu.sync_copy(
            idx_hbm.at[
                wid // workers_per_row,
                pl.ds((wid % workers_per_row) * rows_per_worker, rows_per_worker),
            ],
            idx_v,
        )

        def start_gather(c, b):
            gh = CHUNK // 2
            for h in range(2):
                pltpu.async_copy(
                    table_hbm.at[idx_v.at[pl.ds(c * CHUNK + h * gh, gh)]],
                    bufs.at[b, pl.ds(h * gh, gh)],
                    gsems.at[b],
                )

        def wait_gather(c, b):
            pltpu.make_async_copy(
                table_hbm.at[idx_v.at[pl.ds(c * CHUNK, CHUNK)]],
                bufs.at[b],
                gsems.at[b],
            ).wait()

        def start_wb(c, b):
            pltpu.async_copy(
                bufs.at[b],
                out_hbm.at[pl.ds(base + c * CHUNK, CHUNK)],
                wsems.at[b],
            )

        def wait_wb(c, b):
            pltpu.make_async_copy(
                bufs.at[b],
                out_hbm.at[pl.ds(base + c * CHUNK, CHUNK)],
                wsems.at[b],
            ).wait()

        @pl.loop(0, AHEAD)
        def _(c):
            start_gather(c, c)

        @pl.loop(0, num_chunks)
        def _(c):
            b = lax.rem(c, NBUF)
            wait_gather(c, b)
            nc = c + AHEAD

            @pl.when(nc < num_chunks)
            def _():
                b2 = lax.rem(nc, NBUF)

                @pl.when(nc >= NBUF)
                def _():
                    wait_wb(nc - NBUF, b2)

                start_gather(nc, b2)

            half = CHUNK // 2
            for h in range(2):

                @pl.loop(h * half, (h + 1) * half)
                def _(r):
                    for c0 in range(0, D_MODEL, LANES):
                        bufs[b, r, pl.ds(c0, LANES)] = (
                            bufs[b, r, pl.ds(c0, LANES)] * scale
                        )

                pltpu.async_copy(
                    bufs.at[b, pl.ds(h * half, half)],
                    out_hbm.at[pl.ds(base + c * CHUNK + h * half, half)],
                    wsems.at[b],
                )

        @pl.loop(num_chunks - NBUF, num_chunks)
        def _(c):
            wait_wb(c, lax.rem(c, NBUF))

    return emb_kernel


@jax.jit
def kernel(x, table):
    b, s = x.shape
    out = _make_embedding_kernel(b, s)(table, x.astype(jnp.int32))
    return out.reshape(b, s, D_MODEL)
